# baseline (speedup 1.0000x reference)
import numpy as np

import jax
import jax.numpy as jnp
from jax.experimental import pallas as pl
from jax.experimental.pallas import tpu as pltpu


def _chan_expand(c_in, T):
    r = np.zeros((c_in, c_in * T), np.float32)
    for c in range(c_in * T):
        r[c // T, c] = 1.0
    return r


def _band_mask(Kt, T, CT):
    e = np.zeros((Kt, T, CT), np.float32)
    for k in range(Kt):
        for t in range(T):
            if t + k < T:
                for c in range(t + k, CT, T):
                    e[k, t, c] = 1.0
    return e


def _make_body(b_tile, c_in, c_out, T, T_out, V):
    CT = c_in * T

    def _body(x_ref, lw_ref, wb_ref, cb_ref, lb_ref, o_ref):
        lw = lw_ref[...]
        lwb = lw.astype(jnp.bfloat16)
        wb = wb_ref[...]
        s_col = jax.lax.dot_general(
            lw, jnp.ones((V, 1), jnp.float32),
            dimension_numbers=(((1,), (0,)), ((), ())),
            preferred_element_type=jnp.float32)
        bias = jax.lax.dot_general(
            cb_ref[...], s_col,
            dimension_numbers=(((0,), (1,)), ((), ())),
            preferred_element_type=jnp.float32) + lb_ref[...]
        bias_all = jnp.concatenate([bias] * b_tile, axis=1)
        xws = []
        for b in range(b_tile):
            xb = x_ref[b].reshape(CT, V).astype(jnp.bfloat16)
            xw = jax.lax.dot_general(
                xb, lwb,
                dimension_numbers=(((1,), (1,)), ((), ())),
                preferred_element_type=jnp.float32)
            xws.append(xw.astype(jnp.bfloat16))
        xw_all = jnp.concatenate(xws, axis=1)
        y_all = jax.lax.dot_general(
            wb, xw_all,
            dimension_numbers=(((1,), (0,)), ((), ())),
            preferred_element_type=jnp.float32)
        y3 = y_all.reshape(c_out, T, b_tile * V) + bias_all[:, None, :]
        for b in range(b_tile):
            o_ref[b] = y3[:, :T_out, b * V:(b + 1) * V]
    return _body


def kernel(x, conv_w, conv_b, lin_w, lin_b):
    B, c_in, T, V = x.shape
    c_out, _, Kt, _ = conv_w.shape
    T_out = T - Kt + 1

    rx = jnp.asarray(_chan_expand(c_in, T), jnp.bfloat16)
    e3 = jnp.asarray(_band_mask(Kt, T, c_in * T), jnp.bfloat16)
    w_exp = jnp.einsum('oik,ic->okc', conv_w[:, :, :, 0].astype(jnp.bfloat16),
                       rx, preferred_element_type=jnp.bfloat16)
    wb = jnp.einsum('okc,ktc->otc', w_exp, e3,
                    preferred_element_type=jnp.bfloat16
                    ).reshape(c_out * T, c_in * T)

    cb = conv_b.reshape(1, c_out)
    lb = lin_b.reshape(1, V)

    b_tile = 16
    while B % b_tile:
        b_tile //= 2
    grid = (B // b_tile,)

    return pl.pallas_call(
        _make_body(b_tile, c_in, c_out, T, T_out, V),
        out_shape=jax.ShapeDtypeStruct((B, c_out, T_out, V), jnp.float32),
        grid=grid,
        in_specs=[
            pl.BlockSpec((b_tile, c_in, T, V), lambda g: (g, 0, 0, 0)),
            pl.BlockSpec((V, V), lambda g: (0, 0)),
            pl.BlockSpec((c_out * T, c_in * T), lambda g: (0, 0)),
            pl.BlockSpec((1, c_out), lambda g: (0, 0)),
            pl.BlockSpec((1, V), lambda g: (0, 0)),
        ],
        out_specs=pl.BlockSpec((b_tile, c_out, T_out, V),
                               lambda g: (g, 0, 0, 0)),
        compiler_params=pltpu.CompilerParams(
            dimension_semantics=("parallel",),
            vmem_limit_bytes=64 * 1024 * 1024),
    )(x, lin_w, wb, cb, lb)


# speedup vs baseline: 1.0569x; 1.0569x over previous
import numpy as np

import jax
import jax.numpy as jnp
from jax.experimental import pallas as pl
from jax.experimental.pallas import tpu as pltpu


def _band_selector(Kt, c_in, T):
    e = np.zeros((Kt, T, c_in, c_in * T), np.float32)
    for k in range(Kt):
        for t in range(T):
            if t + k < T:
                for i in range(c_in):
                    e[k, t, i, i * T + t + k] = 1.0
    return e


def _make_body(b_tile, c_in, c_out, T, T_out, V):
    CT = c_in * T

    def _body(x_ref, lw_ref, wb_ref, cb_ref, lb_ref, o_ref):
        lw = lw_ref[...]
        lwb = lw.astype(jnp.bfloat16)
        wb = wb_ref[...]
        s_col = jax.lax.dot_general(
            lw, jnp.ones((V, 1), jnp.float32),
            dimension_numbers=(((1,), (0,)), ((), ())),
            preferred_element_type=jnp.float32)
        bias = jax.lax.dot_general(
            cb_ref[...], s_col,
            dimension_numbers=(((0,), (1,)), ((), ())),
            preferred_element_type=jnp.float32) + lb_ref[...]
        bias_all = jnp.concatenate([bias] * b_tile, axis=1)
        xws = []
        for b in range(b_tile):
            xb = x_ref[b].reshape(CT, V).astype(jnp.bfloat16)
            xw = jax.lax.dot_general(
                xb, lwb,
                dimension_numbers=(((1,), (1,)), ((), ())),
                preferred_element_type=jnp.float32)
            xws.append(xw.astype(jnp.bfloat16))
        xw_all = jnp.concatenate(xws, axis=1)
        y_all = jax.lax.dot_general(
            wb, xw_all,
            dimension_numbers=(((1,), (0,)), ((), ())),
            preferred_element_type=jnp.float32)
        y3 = y_all.reshape(c_out, T, b_tile * V) + bias_all[:, None, :]
        for b in range(b_tile):
            o_ref[b] = y3[:, :T_out, b * V:(b + 1) * V]
    return _body


def kernel(x, conv_w, conv_b, lin_w, lin_b):
    B, c_in, T, V = x.shape
    c_out, _, Kt, _ = conv_w.shape
    T_out = T - Kt + 1

    e4 = jnp.asarray(_band_selector(Kt, c_in, T), jnp.bfloat16)
    wb = jnp.einsum('oik,ktic->otc', conv_w[:, :, :, 0].astype(jnp.bfloat16),
                    e4, preferred_element_type=jnp.bfloat16
                    ).reshape(c_out * T, c_in * T)

    cb = conv_b.reshape(1, c_out)
    lb = lin_b.reshape(1, V)

    b_tile = 16
    while B % b_tile:
        b_tile //= 2
    grid = (B // b_tile,)

    return pl.pallas_call(
        _make_body(b_tile, c_in, c_out, T, T_out, V),
        out_shape=jax.ShapeDtypeStruct((B, c_out, T_out, V), jnp.float32),
        grid=grid,
        in_specs=[
            pl.BlockSpec((b_tile, c_in, T, V), lambda g: (g, 0, 0, 0)),
            pl.BlockSpec((V, V), lambda g: (0, 0)),
            pl.BlockSpec((c_out * T, c_in * T), lambda g: (0, 0)),
            pl.BlockSpec((1, c_out), lambda g: (0, 0)),
            pl.BlockSpec((1, V), lambda g: (0, 0)),
        ],
        out_specs=pl.BlockSpec((b_tile, c_out, T_out, V),
                               lambda g: (g, 0, 0, 0)),
        compiler_params=pltpu.CompilerParams(
            dimension_semantics=("parallel",),
            vmem_limit_bytes=64 * 1024 * 1024),
    )(x, lin_w, wb, cb, lb)
